# baseline (speedup 1.0000x reference)
import functools

import jax
import jax.numpy as jnp
from jax import lax
from jax.experimental import pallas as pl
from jax.experimental.pallas import tpu as pltpu
from jax.experimental.pallas import tpu_sc as plsc

N_NODES = 10000
N_EDGES = 320000
D = 128

NC = 2
NS = 16
NW = NC * NS
LANE = 128
KJ = 80
TS = NW * KJ
EP = TS * LANE
K0 = 80
K1 = 80
CH = 8
NCH0 = K0 // CH
NCH1 = K1 // CH
ACC = 10112
RPT = ACC // NS
ACC_H = 10240
RPT_H = ACC_H // NS

@functools.cache
def _sc_mesh():
    return plsc.VectorSubcoreMesh(
        core_axis_name="c", subcore_axis_name="s",
        num_cores=NC, num_subcores=NS)


def _hist_body(dst_hbm, zeros1_hbm, out_hbm, dst_vm, ones_vm, hist_sh):
    c = lax.axis_index("c")
    s = lax.axis_index("s")
    wid = c * NS + s
    pltpu.sync_copy(zeros1_hbm, hist_sh.at[pl.ds(s * RPT_H, RPT_H)])
    pltpu.sync_copy(dst_hbm.at[pl.ds(pl.multiple_of(wid * KJ, 8), KJ)], dst_vm)
    for k in range(LANE // 16):
        ones_vm[pl.ds(k * 16, 16)] = jnp.ones((16,), jnp.float32)
    plsc.subcore_barrier()

    def body(j, _):
        pltpu.sync_copy(ones_vm, hist_sh.at[dst_vm.at[j]], add=True)
        return ()

    lax.fori_loop(0, KJ, body, ())
    plsc.subcore_barrier()
    pltpu.sync_copy(hist_sh.at[pl.ds(s * RPT_H, RPT_H)],
                    out_hbm.at[c, pl.ds(s * RPT_H, RPT_H)])


@functools.cache
def _sc_hist():
    return pl.kernel(
        _hist_body,
        out_type=jax.ShapeDtypeStruct((NC, ACC_H), jnp.float32),
        mesh=_sc_mesh(),
        scratch_types=[
            pltpu.VMEM((KJ, LANE), jnp.int32),
            pltpu.VMEM((LANE,), jnp.float32),
            pltpu.VMEM_SHARED((ACC_H,), jnp.float32),
        ],
    )


def _agg_body(hs_hbm, src_hbm, dst_hbm, zeros2_hbm, out_hbm,
              sc0_vm, sc1_vm, dc0_vm, dc1_vm, rows0_vm, rows1_vm, acc_sh,
              csem0, csem1, gsem0, gsem1):
    c = lax.axis_index("c")
    s = lax.axis_index("s")
    ibufs = ((sc0_vm, dc0_vm, csem0), (sc1_vm, dc1_vm, csem1))
    gbufs = ((rows0_vm, gsem0), (rows1_vm, gsem1))
    base = pl.multiple_of(
        jnp.where(c == 0, s * K0, NS * K0 + s * K1).astype(jnp.int32), CH)

    def stage(ci, parity):
        sn_vm, dn_vm, nsem = ibufs[parity]
        o = pl.multiple_of(base + ci * CH, CH)
        pltpu.async_copy(src_hbm.at[c, pl.ds(o, CH)], sn_vm, nsem)
        pltpu.async_copy(dst_hbm.at[pl.ds(o, CH)], dn_vm, nsem)

    def chunk(ci):
        sc_vm, dc_vm, csem = ibufs[ci % 2]
        o = pl.multiple_of(base + ci * CH, CH)
        pltpu.make_async_copy(src_hbm.at[c, pl.ds(o, CH)], sc_vm, csem).wait()
        pltpu.make_async_copy(dst_hbm.at[pl.ds(o, CH)], dc_vm, csem).wait()
        if ci + 1 < NCH1:
            stage(ci + 1, (ci + 1) % 2)
        elif ci + 1 < NCH0:
            @pl.when(c == 0)
            def _stage_next():
                stage(ci + 1, (ci + 1) % 2)

        pltpu.async_copy(hs_hbm.at[sc_vm.at[0]], rows0_vm, gsem0)

        @pl.loop(0, CH, step=2)
        def _pipeline(j):
            for b in range(2):
                rows_b, gsem_b = gbufs[b]
                rows_n, gsem_n = gbufs[1 - b]
                jj = j + b
                pltpu.make_async_copy(hs_hbm.at[sc_vm.at[jj]], rows_b, gsem_b).wait()

                @pl.when(jj + 1 < CH)
                def _start_next():
                    pltpu.async_copy(hs_hbm.at[sc_vm.at[jj + 1]], rows_n, gsem_n)

                pltpu.sync_copy(rows_b, acc_sh.at[dc_vm.at[jj]], add=True)

    stage(0, 0)
    pltpu.sync_copy(zeros2_hbm, acc_sh.at[pl.ds(s * RPT, RPT)])
    plsc.subcore_barrier()

    for ci in range(NCH0):
        if ci < NCH1:
            chunk(ci)
        else:
            @pl.when(c == 0)
            def _chunk_c0():
                chunk(ci)

    plsc.subcore_barrier()
    pltpu.sync_copy(acc_sh.at[pl.ds(s * RPT, RPT)],
                    out_hbm.at[c, pl.ds(s * RPT, RPT)])


@functools.cache
def _sc_agg():
    return pl.kernel(
        _agg_body,
        out_type=jax.ShapeDtypeStruct((NC, ACC, D), jnp.float32),
        mesh=_sc_mesh(),
        scratch_types=[
            pltpu.VMEM((CH, LANE), jnp.int32),
            pltpu.VMEM((CH, LANE), jnp.int32),
            pltpu.VMEM((CH, LANE), jnp.int32),
            pltpu.VMEM((CH, LANE), jnp.int32),
            pltpu.VMEM((LANE, D), jnp.float32),
            pltpu.VMEM((LANE, D), jnp.float32),
            pltpu.VMEM_SHARED((ACC, D), jnp.float32),
            pltpu.SemaphoreType.DMA,
            pltpu.SemaphoreType.DMA,
            pltpu.SemaphoreType.DMA,
            pltpu.SemaphoreType.DMA,
        ],
    )


def _mm_scale_body(x_ref, w_ref, h0_ref, h1_ref, o_ref):
    dinv = lax.rsqrt(h0_ref[...] + h1_ref[...] + 1.0)
    o_ref[...] = jnp.dot(x_ref[...], w_ref[...],
                         preferred_element_type=jnp.float32) * dinv


def _mid_body(p0_ref, p1_ref, hs_ref, h0_ref, h1_ref, b_ref, w_ref, o_ref):
    dinv = lax.rsqrt(h0_ref[...] + h1_ref[...] + 1.0)
    t = (p0_ref[...] + p1_ref[...] + hs_ref[...]) * dinv + b_ref[...]
    t = jnp.maximum(t, 0.0)
    o_ref[...] = jnp.dot(t, w_ref[...],
                         preferred_element_type=jnp.float32) * dinv


def _final_body(q0_ref, q1_ref, hs_ref, h0_ref, h1_ref, b_ref, o_ref):
    dinv = lax.rsqrt(h0_ref[...] + h1_ref[...] + 1.0)
    z = (q0_ref[...] + q1_ref[...] + hs_ref[...]) * dinv + b_ref[...]
    m = jnp.max(z, axis=1, keepdims=True)
    lse = jnp.log(jnp.sum(jnp.exp(z - m), axis=1, keepdims=True)) + m
    o_ref[...] = z - lse


_BLK_A = ACC // 16
ACC2 = 2 * ACC


def _tc_mm_scale(xp, W, h0, h1):
    return pl.pallas_call(
        _mm_scale_body,
        grid=(32,),
        in_specs=[
            pl.BlockSpec((_BLK_A, D), lambda i: (i % 16, 0)),
            pl.BlockSpec((D, D), lambda i: (0, 0)),
            pl.BlockSpec((_BLK_A, 1), lambda i: (i % 16, 0)),
            pl.BlockSpec((_BLK_A, 1), lambda i: (i % 16, 0)),
        ],
        out_specs=pl.BlockSpec((_BLK_A, D), lambda i: (i, 0)),
        out_shape=jax.ShapeDtypeStruct((ACC2, D), jnp.float32),
    )(xp, W, h0, h1)


def _tc_mid(p0, p1, hs, h0, h1, b, W):
    return pl.pallas_call(
        _mid_body,
        grid=(32,),
        in_specs=[
            pl.BlockSpec((_BLK_A, D), lambda i: (i % 16, 0)),
            pl.BlockSpec((_BLK_A, D), lambda i: (i % 16, 0)),
            pl.BlockSpec((_BLK_A, D), lambda i: (i % 16, 0)),
            pl.BlockSpec((_BLK_A, 1), lambda i: (i % 16, 0)),
            pl.BlockSpec((_BLK_A, 1), lambda i: (i % 16, 0)),
            pl.BlockSpec((1, D), lambda i: (0, 0)),
            pl.BlockSpec((D, D), lambda i: (0, 0)),
        ],
        out_specs=pl.BlockSpec((_BLK_A, D), lambda i: (i, 0)),
        out_shape=jax.ShapeDtypeStruct((ACC2, D), jnp.float32),
    )(p0, p1, hs, h0, h1, b, W)


_BLK_C = 400


def _tc_final(q0, q1, hs, h0, h1, b):
    return pl.pallas_call(
        _final_body,
        grid=(N_NODES // _BLK_C,),
        in_specs=[
            pl.BlockSpec((_BLK_C, D), lambda i: (i, 0)),
            pl.BlockSpec((_BLK_C, D), lambda i: (i, 0)),
            pl.BlockSpec((_BLK_C, D), lambda i: (i, 0)),
            pl.BlockSpec((_BLK_C, 1), lambda i: (i, 0)),
            pl.BlockSpec((_BLK_C, 1), lambda i: (i, 0)),
            pl.BlockSpec((1, D), lambda i: (0, 0)),
        ],
        out_specs=pl.BlockSpec((_BLK_C, D), lambda i: (i, 0)),
        out_shape=jax.ShapeDtypeStruct((N_NODES, D), jnp.float32),
    )(q0, q1, hs, h0, h1, b)


def kernel(x, edge_index, W1, b1, W2, b2):
    src = edge_index[0]
    dst = edge_index[1]
    pad = EP - N_EDGES
    srcp = jnp.concatenate(
        [src, jnp.zeros((pad,), jnp.int32)]).reshape(TS, LANE)
    srcp2 = jnp.stack([srcp, srcp + ACC])
    dstp = jnp.concatenate(
        [dst, jnp.full((pad,), N_NODES, jnp.int32)]).reshape(TS, LANE)
    xp = jnp.pad(x, ((0, ACC - N_NODES), (0, 0)))
    zeros1 = jnp.zeros((RPT_H,), jnp.float32)
    zeros2 = jnp.zeros((RPT, D), jnp.float32)

    hist = _sc_hist()(dstp, zeros1)
    h0 = hist[0, :ACC].reshape(ACC, 1)
    h1 = hist[1, :ACC].reshape(ACC, 1)
    b1r = b1.reshape(1, D)
    b2r = b2.reshape(1, D)

    hs1 = _tc_mm_scale(xp, W1, h0, h1)
    p = _sc_agg()(hs1, srcp2, dstp, zeros2)
    hs2 = _tc_mid(p[0], p[1], hs1, h0, h1, b1r, W2)
    q = _sc_agg()(hs2, srcp2, dstp, zeros2)
    return _tc_final(q[0], q[1], hs2, h0, h1, b2r)


# speedup vs baseline: 1.1226x; 1.1226x over previous
import functools

import jax
import jax.numpy as jnp
from jax import lax
from jax.experimental import pallas as pl
from jax.experimental.pallas import tpu as pltpu
from jax.experimental.pallas import tpu_sc as plsc

N_NODES = 10000
N_EDGES = 320000
D = 128

NC = 2
NS = 16
NW = NC * NS
LANE = 128
KJ = 80
TS = NW * KJ
EP = TS * LANE
K0 = 104
K1 = 56
ACC = 10112
RPT = ACC // NS
ACC_H = 10240
RPT_H = ACC_H // NS

@functools.cache
def _sc_mesh():
    return plsc.VectorSubcoreMesh(
        core_axis_name="c", subcore_axis_name="s",
        num_cores=NC, num_subcores=NS)


def _hist_body(dst_hbm, zeros1_hbm, out_hbm, dst_vm, ones_vm, hist_sh):
    c = lax.axis_index("c")
    s = lax.axis_index("s")
    wid = c * NS + s
    pltpu.sync_copy(zeros1_hbm, hist_sh.at[pl.ds(s * RPT_H, RPT_H)])
    pltpu.sync_copy(dst_hbm.at[pl.ds(pl.multiple_of(wid * KJ, 8), KJ)], dst_vm)
    for k in range(LANE // 16):
        ones_vm[pl.ds(k * 16, 16)] = jnp.ones((16,), jnp.float32)
    plsc.subcore_barrier()

    def body(j, _):
        pltpu.sync_copy(ones_vm, hist_sh.at[dst_vm.at[j]], add=True)
        return ()

    lax.fori_loop(0, KJ, body, ())
    plsc.subcore_barrier()
    pltpu.sync_copy(hist_sh.at[pl.ds(s * RPT_H, RPT_H)],
                    out_hbm.at[c, pl.ds(s * RPT_H, RPT_H)])


@functools.cache
def _sc_hist():
    return pl.kernel(
        _hist_body,
        out_type=jax.ShapeDtypeStruct((NC, ACC_H), jnp.float32),
        mesh=_sc_mesh(),
        scratch_types=[
            pltpu.VMEM((KJ, LANE), jnp.int32),
            pltpu.VMEM((LANE,), jnp.float32),
            pltpu.VMEM_SHARED((ACC_H,), jnp.float32),
        ],
    )


def _agg_body(hs_hbm, src_hbm, dst_hbm, zeros2_hbm, out_hbm,
              src_vm, dst_vm, rows_vm, acc_sh, sem):
    c = lax.axis_index("c")
    s = lax.axis_index("s")
    base = pl.multiple_of(
        jnp.where(c == 0, s * K0, NS * K0 + s * K1).astype(jnp.int32), 8)
    n = jnp.where(c == 0, K0, K1)

    @pl.when(c == 0)
    def _stage0():
        pltpu.sync_copy(src_hbm.at[pl.ds(base, K0)], src_vm.at[pl.ds(0, K0)])
        pltpu.sync_copy(dst_hbm.at[pl.ds(base, K0)], dst_vm.at[pl.ds(0, K0)])

    @pl.when(c == 1)
    def _stage1():
        pltpu.sync_copy(src_hbm.at[pl.ds(base, K1)], src_vm.at[pl.ds(0, K1)])
        pltpu.sync_copy(dst_hbm.at[pl.ds(base, K1)], dst_vm.at[pl.ds(0, K1)])

    pltpu.sync_copy(zeros2_hbm, acc_sh.at[pl.ds(s * RPT, RPT)])
    plsc.subcore_barrier()

    def body(j, _):
        pltpu.async_copy(hs_hbm.at[src_vm.at[j]], rows_vm, sem).wait()
        pltpu.sync_copy(rows_vm, acc_sh.at[dst_vm.at[j]], add=True)
        return ()

    lax.fori_loop(0, n, body, ())
    plsc.subcore_barrier()
    pltpu.sync_copy(acc_sh.at[pl.ds(s * RPT, RPT)],
                    out_hbm.at[c, pl.ds(s * RPT, RPT)])


@functools.cache
def _sc_agg():
    return pl.kernel(
        _agg_body,
        out_type=jax.ShapeDtypeStruct((NC, ACC, D), jnp.float32),
        mesh=_sc_mesh(),
        scratch_types=[
            pltpu.VMEM((K0, LANE), jnp.int32),
            pltpu.VMEM((K0, LANE), jnp.int32),
            pltpu.VMEM((LANE, D), jnp.float32),
            pltpu.VMEM_SHARED((ACC, D), jnp.float32),
            pltpu.SemaphoreType.DMA,
        ],
    )


def _mm_scale_body(x_ref, w_ref, h0_ref, h1_ref, o_ref):
    dinv = lax.rsqrt(h0_ref[...] + h1_ref[...] + 1.0)
    o_ref[...] = jnp.dot(x_ref[...], w_ref[...],
                         preferred_element_type=jnp.float32) * dinv


def _mid_body(p0_ref, p1_ref, hs_ref, h0_ref, h1_ref, b_ref, w_ref, o_ref):
    dinv = lax.rsqrt(h0_ref[...] + h1_ref[...] + 1.0)
    t = (p0_ref[...] + p1_ref[...] + hs_ref[...]) * dinv + b_ref[...]
    t = jnp.maximum(t, 0.0)
    o_ref[...] = jnp.dot(t, w_ref[...],
                         preferred_element_type=jnp.float32) * dinv


def _final_body(q0_ref, q1_ref, hs_ref, h0_ref, h1_ref, b_ref, o_ref):
    dinv = lax.rsqrt(h0_ref[...] + h1_ref[...] + 1.0)
    z = (q0_ref[...] + q1_ref[...] + hs_ref[...]) * dinv + b_ref[...]
    m = jnp.max(z, axis=1, keepdims=True)
    lse = jnp.log(jnp.sum(jnp.exp(z - m), axis=1, keepdims=True)) + m
    o_ref[...] = z - lse


_BLK_A = ACC // 16


def _tc_mm_scale(xp, W, h0, h1):
    return pl.pallas_call(
        _mm_scale_body,
        grid=(16,),
        in_specs=[
            pl.BlockSpec((_BLK_A, D), lambda i: (i, 0)),
            pl.BlockSpec((D, D), lambda i: (0, 0)),
            pl.BlockSpec((_BLK_A, 1), lambda i: (i, 0)),
            pl.BlockSpec((_BLK_A, 1), lambda i: (i, 0)),
        ],
        out_specs=pl.BlockSpec((_BLK_A, D), lambda i: (i, 0)),
        out_shape=jax.ShapeDtypeStruct((ACC, D), jnp.float32),
    )(xp, W, h0, h1)


def _tc_mid(p0, p1, hs, h0, h1, b, W):
    return pl.pallas_call(
        _mid_body,
        grid=(16,),
        in_specs=[
            pl.BlockSpec((_BLK_A, D), lambda i: (i, 0)),
            pl.BlockSpec((_BLK_A, D), lambda i: (i, 0)),
            pl.BlockSpec((_BLK_A, D), lambda i: (i, 0)),
            pl.BlockSpec((_BLK_A, 1), lambda i: (i, 0)),
            pl.BlockSpec((_BLK_A, 1), lambda i: (i, 0)),
            pl.BlockSpec((1, D), lambda i: (0, 0)),
            pl.BlockSpec((D, D), lambda i: (0, 0)),
        ],
        out_specs=pl.BlockSpec((_BLK_A, D), lambda i: (i, 0)),
        out_shape=jax.ShapeDtypeStruct((ACC, D), jnp.float32),
    )(p0, p1, hs, h0, h1, b, W)


_BLK_C = 400


def _tc_final(q0, q1, hs, h0, h1, b):
    return pl.pallas_call(
        _final_body,
        grid=(N_NODES // _BLK_C,),
        in_specs=[
            pl.BlockSpec((_BLK_C, D), lambda i: (i, 0)),
            pl.BlockSpec((_BLK_C, D), lambda i: (i, 0)),
            pl.BlockSpec((_BLK_C, D), lambda i: (i, 0)),
            pl.BlockSpec((_BLK_C, 1), lambda i: (i, 0)),
            pl.BlockSpec((_BLK_C, 1), lambda i: (i, 0)),
            pl.BlockSpec((1, D), lambda i: (0, 0)),
        ],
        out_specs=pl.BlockSpec((_BLK_C, D), lambda i: (i, 0)),
        out_shape=jax.ShapeDtypeStruct((N_NODES, D), jnp.float32),
    )(q0, q1, hs, h0, h1, b)


def kernel(x, edge_index, W1, b1, W2, b2):
    src = edge_index[0]
    dst = edge_index[1]
    pad = EP - N_EDGES
    srcp = jnp.concatenate(
        [src, jnp.zeros((pad,), jnp.int32)]).reshape(TS, LANE)
    dstp = jnp.concatenate(
        [dst, jnp.full((pad,), N_NODES, jnp.int32)]).reshape(TS, LANE)
    xp = jnp.pad(x, ((0, ACC - N_NODES), (0, 0)))
    zeros1 = jnp.zeros((RPT_H,), jnp.float32)
    zeros2 = jnp.zeros((RPT, D), jnp.float32)

    hist = _sc_hist()(dstp, zeros1)
    h0 = hist[0, :ACC].reshape(ACC, 1)
    h1 = hist[1, :ACC].reshape(ACC, 1)
    b1r = b1.reshape(1, D)
    b2r = b2.reshape(1, D)

    hs1 = _tc_mm_scale(xp, W1, h0, h1)
    p = _sc_agg()(hs1, srcp, dstp, zeros2)
    hs2 = _tc_mid(p[0], p[1], hs1, h0, h1, b1r, W2)
    q = _sc_agg()(hs2, srcp, dstp, zeros2)
    return _tc_final(q[0], q[1], hs2, h0, h1, b2r)


# speedup vs baseline: 1.3710x; 1.2212x over previous
import functools

import jax
import jax.numpy as jnp
from jax import lax
from jax.experimental import pallas as pl
from jax.experimental.pallas import tpu as pltpu
from jax.experimental.pallas import tpu_sc as plsc

N_NODES = 10000
N_EDGES = 320000
D = 128

NC = 2
NS = 16
NW = NC * NS
LANE = 128
KJ = 79
EP = NW * KJ * LANE
ACC = 10240
RPT = ACC // NS

@functools.cache
def _sc_mesh():
    return plsc.VectorSubcoreMesh(
        core_axis_name="c", subcore_axis_name="s",
        num_cores=NC, num_subcores=NS)


def _hist_body(dst_hbm, zeros1_hbm, out_hbm, dst_vm, ones_vm, hist_sh):
    c = lax.axis_index("c")
    s = lax.axis_index("s")
    wid = c * NS + s
    pltpu.sync_copy(zeros1_hbm, hist_sh.at[pl.ds(s * RPT, RPT)])
    pltpu.sync_copy(dst_hbm.at[wid], dst_vm)
    for k in range(LANE // 16):
        ones_vm[pl.ds(k * 16, 16)] = jnp.ones((16,), jnp.float32)
    plsc.subcore_barrier()

    def body(j, _):
        pltpu.sync_copy(ones_vm, hist_sh.at[dst_vm.at[j]], add=True)
        return ()

    lax.fori_loop(0, KJ, body, ())
    plsc.subcore_barrier()
    pltpu.sync_copy(hist_sh.at[pl.ds(s * RPT, RPT)],
                    out_hbm.at[c, pl.ds(s * RPT, RPT)])


@functools.cache
def _sc_hist():
    return pl.kernel(
        _hist_body,
        out_type=jax.ShapeDtypeStruct((NC, ACC), jnp.float32),
        mesh=_sc_mesh(),
        scratch_types=[
            pltpu.VMEM((KJ, LANE), jnp.int32),
            pltpu.VMEM((LANE,), jnp.float32),
            pltpu.VMEM_SHARED((ACC,), jnp.float32),
        ],
    )


def _agg_body(hs_hbm, src_hbm, dst_hbm, zeros2_hbm, out_hbm,
              src_vm, dst_vm, rows_vm, acc_sh, sem):
    c = lax.axis_index("c")
    s = lax.axis_index("s")
    wid = c * NS + s
    pltpu.sync_copy(zeros2_hbm, acc_sh.at[pl.ds(s * RPT, RPT)])
    pltpu.sync_copy(src_hbm.at[wid], src_vm)
    pltpu.sync_copy(dst_hbm.at[wid], dst_vm)
    plsc.subcore_barrier()

    def body(j, _):
        pltpu.async_copy(hs_hbm.at[src_vm.at[j]], rows_vm, sem).wait()
        pltpu.sync_copy(rows_vm, acc_sh.at[dst_vm.at[j]], add=True)
        return ()

    lax.fori_loop(0, KJ, body, ())
    plsc.subcore_barrier()
    pltpu.sync_copy(acc_sh.at[pl.ds(s * RPT, RPT)],
                    out_hbm.at[c, pl.ds(s * RPT, RPT)])


@functools.cache
def _sc_agg():
    return pl.kernel(
        _agg_body,
        out_type=jax.ShapeDtypeStruct((NC, ACC, D), jnp.float32),
        mesh=_sc_mesh(),
        scratch_types=[
            pltpu.VMEM((KJ, LANE), jnp.int32),
            pltpu.VMEM((KJ, LANE), jnp.int32),
            pltpu.VMEM((LANE, D), jnp.float32),
            pltpu.VMEM_SHARED((ACC, D), jnp.float32),
            pltpu.SemaphoreType.DMA,
        ],
    )


def _mm_scale_body(x_ref, w_ref, h0_ref, h1_ref, o_ref):
    dinv = lax.rsqrt(h0_ref[...] + h1_ref[...] + 1.0)
    o_ref[...] = jnp.dot(x_ref[...], w_ref[...],
                         preferred_element_type=jnp.float32) * dinv


def _mid_body(p0_ref, p1_ref, hs_ref, h0_ref, h1_ref, b_ref, w_ref, o_ref):
    dinv = lax.rsqrt(h0_ref[...] + h1_ref[...] + 1.0)
    t = (p0_ref[...] + p1_ref[...] + hs_ref[...]) * dinv + b_ref[...]
    t = jnp.maximum(t, 0.0)
    o_ref[...] = jnp.dot(t, w_ref[...],
                         preferred_element_type=jnp.float32) * dinv


def _final_body(q0_ref, q1_ref, hs_ref, h0_ref, h1_ref, b_ref, o_ref):
    dinv = lax.rsqrt(h0_ref[...] + h1_ref[...] + 1.0)
    z = (q0_ref[...] + q1_ref[...] + hs_ref[...]) * dinv + b_ref[...]
    m = jnp.max(z, axis=1, keepdims=True)
    lse = jnp.log(jnp.sum(jnp.exp(z - m), axis=1, keepdims=True)) + m
    o_ref[...] = z - lse


_BLK_A = ACC // 16


def _tc_mm_scale(xp, W, h0, h1):
    return pl.pallas_call(
        _mm_scale_body,
        grid=(16,),
        in_specs=[
            pl.BlockSpec((_BLK_A, D), lambda i: (i, 0)),
            pl.BlockSpec((D, D), lambda i: (0, 0)),
            pl.BlockSpec((_BLK_A, 1), lambda i: (i, 0)),
            pl.BlockSpec((_BLK_A, 1), lambda i: (i, 0)),
        ],
        out_specs=pl.BlockSpec((_BLK_A, D), lambda i: (i, 0)),
        out_shape=jax.ShapeDtypeStruct((ACC, D), jnp.float32),
    )(xp, W, h0, h1)


def _tc_mid(p0, p1, hs, h0, h1, b, W):
    return pl.pallas_call(
        _mid_body,
        grid=(16,),
        in_specs=[
            pl.BlockSpec((_BLK_A, D), lambda i: (i, 0)),
            pl.BlockSpec((_BLK_A, D), lambda i: (i, 0)),
            pl.BlockSpec((_BLK_A, D), lambda i: (i, 0)),
            pl.BlockSpec((_BLK_A, 1), lambda i: (i, 0)),
            pl.BlockSpec((_BLK_A, 1), lambda i: (i, 0)),
            pl.BlockSpec((1, D), lambda i: (0, 0)),
            pl.BlockSpec((D, D), lambda i: (0, 0)),
        ],
        out_specs=pl.BlockSpec((_BLK_A, D), lambda i: (i, 0)),
        out_shape=jax.ShapeDtypeStruct((ACC, D), jnp.float32),
    )(p0, p1, hs, h0, h1, b, W)


_BLK_C = 400


def _tc_final(q0, q1, hs, h0, h1, b):
    return pl.pallas_call(
        _final_body,
        grid=(N_NODES // _BLK_C,),
        in_specs=[
            pl.BlockSpec((_BLK_C, D), lambda i: (i, 0)),
            pl.BlockSpec((_BLK_C, D), lambda i: (i, 0)),
            pl.BlockSpec((_BLK_C, D), lambda i: (i, 0)),
            pl.BlockSpec((_BLK_C, 1), lambda i: (i, 0)),
            pl.BlockSpec((_BLK_C, 1), lambda i: (i, 0)),
            pl.BlockSpec((1, D), lambda i: (0, 0)),
        ],
        out_specs=pl.BlockSpec((_BLK_C, D), lambda i: (i, 0)),
        out_shape=jax.ShapeDtypeStruct((N_NODES, D), jnp.float32),
    )(q0, q1, hs, h0, h1, b)


def kernel(x, edge_index, W1, b1, W2, b2):
    src = edge_index[0]
    dst = edge_index[1]
    pad = EP - N_EDGES
    srcp = jnp.concatenate(
        [src, jnp.zeros((pad,), jnp.int32)]).reshape(NW, KJ, LANE)
    dstp = jnp.concatenate(
        [dst, jnp.full((pad,), N_NODES, jnp.int32)]).reshape(NW, KJ, LANE)
    xp = jnp.pad(x, ((0, ACC - N_NODES), (0, 0)))
    zeros1 = jnp.zeros((RPT,), jnp.float32)
    zeros2 = jnp.zeros((RPT, D), jnp.float32)

    hist = _sc_hist()(dstp, zeros1)
    h0 = hist[0].reshape(ACC, 1)
    h1 = hist[1].reshape(ACC, 1)
    b1r = b1.reshape(1, D)
    b2r = b2.reshape(1, D)

    hs1 = _tc_mm_scale(xp, W1, h0, h1)
    p = _sc_agg()(hs1, srcp, dstp, zeros2)
    hs2 = _tc_mid(p[0], p[1], hs1, h0, h1, b1r, W2)
    q = _sc_agg()(hs2, srcp, dstp, zeros2)
    return _tc_final(q[0], q[1], hs2, h0, h1, b2r)


# speedup vs baseline: 1.3943x; 1.0170x over previous
import functools

import jax
import jax.numpy as jnp
from jax import lax
from jax.experimental import pallas as pl
from jax.experimental.pallas import tpu as pltpu
from jax.experimental.pallas import tpu_sc as plsc

N_NODES = 10000
N_EDGES = 320000
D = 128

NC = 2
NS = 16
NW = NC * NS
LANE = 128
KJ = 79
EP = NW * KJ * LANE
ACC = 10240
RPT = ACC // NS

@functools.cache
def _sc_mesh():
    return plsc.VectorSubcoreMesh(
        core_axis_name="c", subcore_axis_name="s",
        num_cores=NC, num_subcores=NS)


def _hist_body(dst_hbm, zeros1_hbm, out_hbm, dst_vm, ones_vm, hist_sh):
    c = lax.axis_index("c")
    s = lax.axis_index("s")
    wid = c * NS + s
    pltpu.sync_copy(zeros1_hbm, hist_sh.at[pl.ds(s * RPT, RPT)])
    pltpu.sync_copy(dst_hbm.at[wid], dst_vm)
    for k in range(LANE // 16):
        ones_vm[pl.ds(k * 16, 16)] = jnp.ones((16,), jnp.float32)
    plsc.subcore_barrier()

    def body(j, _):
        pltpu.sync_copy(ones_vm, hist_sh.at[dst_vm.at[j]], add=True)
        return ()

    lax.fori_loop(0, KJ, body, ())
    plsc.subcore_barrier()
    pltpu.sync_copy(hist_sh.at[pl.ds(s * RPT, RPT)],
                    out_hbm.at[c, pl.ds(s * RPT, RPT)])


@functools.cache
def _sc_hist():
    return pl.kernel(
        _hist_body,
        out_type=jax.ShapeDtypeStruct((NC, ACC), jnp.float32),
        mesh=_sc_mesh(),
        scratch_types=[
            pltpu.VMEM((KJ, LANE), jnp.int32),
            pltpu.VMEM((LANE,), jnp.float32),
            pltpu.VMEM_SHARED((ACC,), jnp.float32),
        ],
    )


def _agg_body(hs_hbm, src_hbm, dst_hbm, out_hbm,
              src_vm, dst_vm, rows_vm, acc_sh, sem):
    c = lax.axis_index("c")
    s = lax.axis_index("s")
    wid = c * NS + s
    d_src = pltpu.async_copy(src_hbm.at[wid], src_vm, sem)
    d_dst = pltpu.async_copy(dst_hbm.at[wid], dst_vm, sem)

    @pl.loop(0, LANE)
    def _zfill(r):
        for k in range(D // 16):
            rows_vm[r, pl.ds(k * 16, 16)] = jnp.zeros((16,), jnp.float32)

    for t in range(RPT // LANE):
        pltpu.sync_copy(rows_vm, acc_sh.at[pl.ds(s * RPT + t * LANE, LANE)])
    d_src.wait()
    d_dst.wait()
    plsc.subcore_barrier()

    def body(j, _):
        pltpu.async_copy(hs_hbm.at[src_vm.at[j]], rows_vm, sem).wait()
        pltpu.sync_copy(rows_vm, acc_sh.at[dst_vm.at[j]], add=True)
        return ()

    lax.fori_loop(0, KJ, body, ())
    plsc.subcore_barrier()
    pltpu.sync_copy(acc_sh.at[pl.ds(s * RPT, RPT)],
                    out_hbm.at[c, pl.ds(s * RPT, RPT)])


@functools.cache
def _sc_agg():
    return pl.kernel(
        _agg_body,
        out_type=jax.ShapeDtypeStruct((NC, ACC, D), jnp.float32),
        mesh=_sc_mesh(),
        scratch_types=[
            pltpu.VMEM((KJ, LANE), jnp.int32),
            pltpu.VMEM((KJ, LANE), jnp.int32),
            pltpu.VMEM((LANE, D), jnp.float32),
            pltpu.VMEM_SHARED((ACC, D), jnp.float32),
            pltpu.SemaphoreType.DMA,
        ],
    )


def _mm_scale_body(x_ref, w_ref, h0_ref, h1_ref, o_ref):
    dinv = lax.rsqrt(h0_ref[...] + h1_ref[...] + 1.0)
    o_ref[...] = jnp.dot(x_ref[...], w_ref[...],
                         preferred_element_type=jnp.float32) * dinv


def _mid_body(p0_ref, p1_ref, hs_ref, h0_ref, h1_ref, b_ref, w_ref, o_ref):
    dinv = lax.rsqrt(h0_ref[...] + h1_ref[...] + 1.0)
    t = (p0_ref[...] + p1_ref[...] + hs_ref[...]) * dinv + b_ref[...]
    t = jnp.maximum(t, 0.0)
    o_ref[...] = jnp.dot(t, w_ref[...],
                         preferred_element_type=jnp.float32) * dinv


def _final_body(q0_ref, q1_ref, hs_ref, h0_ref, h1_ref, b_ref, o_ref):
    dinv = lax.rsqrt(h0_ref[...] + h1_ref[...] + 1.0)
    z = (q0_ref[...] + q1_ref[...] + hs_ref[...]) * dinv + b_ref[...]
    m = jnp.max(z, axis=1, keepdims=True)
    lse = jnp.log(jnp.sum(jnp.exp(z - m), axis=1, keepdims=True)) + m
    o_ref[...] = z - lse


_BLK_A = ACC // 16


def _tc_mm_scale(xp, W, h0, h1):
    return pl.pallas_call(
        _mm_scale_body,
        grid=(16,),
        in_specs=[
            pl.BlockSpec((_BLK_A, D), lambda i: (i, 0)),
            pl.BlockSpec((D, D), lambda i: (0, 0)),
            pl.BlockSpec((_BLK_A, 1), lambda i: (i, 0)),
            pl.BlockSpec((_BLK_A, 1), lambda i: (i, 0)),
        ],
        out_specs=pl.BlockSpec((_BLK_A, D), lambda i: (i, 0)),
        out_shape=jax.ShapeDtypeStruct((ACC, D), jnp.float32),
    )(xp, W, h0, h1)


def _tc_mid(p0, p1, hs, h0, h1, b, W):
    return pl.pallas_call(
        _mid_body,
        grid=(16,),
        in_specs=[
            pl.BlockSpec((_BLK_A, D), lambda i: (i, 0)),
            pl.BlockSpec((_BLK_A, D), lambda i: (i, 0)),
            pl.BlockSpec((_BLK_A, D), lambda i: (i, 0)),
            pl.BlockSpec((_BLK_A, 1), lambda i: (i, 0)),
            pl.BlockSpec((_BLK_A, 1), lambda i: (i, 0)),
            pl.BlockSpec((1, D), lambda i: (0, 0)),
            pl.BlockSpec((D, D), lambda i: (0, 0)),
        ],
        out_specs=pl.BlockSpec((_BLK_A, D), lambda i: (i, 0)),
        out_shape=jax.ShapeDtypeStruct((ACC, D), jnp.float32),
    )(p0, p1, hs, h0, h1, b, W)


_BLK_C = 400


def _tc_final(q0, q1, hs, h0, h1, b):
    return pl.pallas_call(
        _final_body,
        grid=(N_NODES // _BLK_C,),
        in_specs=[
            pl.BlockSpec((_BLK_C, D), lambda i: (i, 0)),
            pl.BlockSpec((_BLK_C, D), lambda i: (i, 0)),
            pl.BlockSpec((_BLK_C, D), lambda i: (i, 0)),
            pl.BlockSpec((_BLK_C, 1), lambda i: (i, 0)),
            pl.BlockSpec((_BLK_C, 1), lambda i: (i, 0)),
            pl.BlockSpec((1, D), lambda i: (0, 0)),
        ],
        out_specs=pl.BlockSpec((_BLK_C, D), lambda i: (i, 0)),
        out_shape=jax.ShapeDtypeStruct((N_NODES, D), jnp.float32),
    )(q0, q1, hs, h0, h1, b)


def kernel(x, edge_index, W1, b1, W2, b2):
    src = edge_index[0]
    dst = edge_index[1]
    pad = EP - N_EDGES
    srcp = jnp.concatenate(
        [src, jnp.zeros((pad,), jnp.int32)]).reshape(NW, KJ, LANE)
    dstp = jnp.concatenate(
        [dst, jnp.full((pad,), N_NODES, jnp.int32)]).reshape(NW, KJ, LANE)
    xp = jnp.pad(x, ((0, ACC - N_NODES), (0, 0)))
    zeros1 = jnp.zeros((RPT,), jnp.float32)

    hist = _sc_hist()(dstp, zeros1)
    h0 = hist[0].reshape(ACC, 1)
    h1 = hist[1].reshape(ACC, 1)
    b1r = b1.reshape(1, D)
    b2r = b2.reshape(1, D)

    hs1 = _tc_mm_scale(xp, W1, h0, h1)
    p = _sc_agg()(hs1, srcp, dstp)
    hs2 = _tc_mid(p[0], p[1], hs1, h0, h1, b1r, W2)
    q = _sc_agg()(hs2, srcp, dstp)
    return _tc_final(q[0], q[1], hs2, h0, h1, b2r)
